# speedup vs baseline: 1.5494x; 1.5494x over previous
import jax
import jax.numpy as jnp
from jax.experimental import pallas as pl
from jax.experimental.pallas import tpu as pltpu

N = 10000
E = 320000
D = 128
H = 128
C = 40


def _k0_body(x_ref, w_ref, b_ref, h_ref, hn_ref):
    h = jnp.maximum(jnp.dot(x_ref[...], w_ref[...],
                            preferred_element_type=jnp.float32) + b_ref[...], 0.0)
    h_ref[...] = h
    n2 = jnp.sum(h * h, axis=1, keepdims=True)
    hn_ref[...] = h / jnp.maximum(jnp.sqrt(n2), 1e-12)


def _dense_in(x, W1, b1):
    return pl.pallas_call(
        _k0_body,
        out_shape=[jax.ShapeDtypeStruct((N, H), jnp.float32),
                   jax.ShapeDtypeStruct((N, H), jnp.float32)],
    )(x, W1, b1.reshape(1, H))


def _norm_body(h_ref, hn_ref):
    h = h_ref[...]
    n2 = jnp.sum(h * h, axis=1, keepdims=True)
    hn_ref[...] = h / jnp.maximum(jnp.sqrt(n2), 1e-12)


def _normalize(h):
    return pl.pallas_call(
        _norm_body,
        out_shape=jax.ShapeDtypeStruct((N, H), jnp.float32),
    )(h)


def _kf_body(h_ref, w_ref, b_ref, o_ref):
    logits = jnp.dot(h_ref[...], w_ref[...],
                     preferred_element_type=jnp.float32) + b_ref[...]
    m = jnp.max(logits, axis=1, keepdims=True)
    z = logits - m
    lse = jnp.log(jnp.sum(jnp.exp(z), axis=1, keepdims=True))
    o_ref[...] = z - lse


def _dense_out(h, W4, b4):
    return pl.pallas_call(
        _kf_body,
        out_shape=jax.ShapeDtypeStruct((N, C), jnp.float32),
    )(h, W4, b4.reshape(1, C))


def _conv_jax(h, hn, beta, src, dst):
    alpha = beta * jnp.sum(hn[src] * hn[dst], axis=-1)
    shift = jnp.abs(beta)
    ex = jnp.exp(alpha - shift)
    denom = jax.ops.segment_sum(ex, dst, num_segments=N)
    coef = ex / (denom[dst] + 1e-16)
    return jax.ops.segment_sum(coef[:, None] * h[src], dst, num_segments=N)


def kernel(x, edge_index, W1, b1, beta2, beta3, beta5, beta6, W4, b4):
    self_loops = jnp.arange(N, dtype=edge_index.dtype)
    src = jnp.concatenate([edge_index[0], self_loops])
    dst = jnp.concatenate([edge_index[1], self_loops])
    h, hn = _dense_in(x, W1, b1)
    for beta in (beta2, beta3, beta5, beta6):
        h = _conv_jax(h, hn, beta[0], src, dst)
        hn = _normalize(h)
    return _dense_out(h, W4, b4)


# speedup vs baseline: 3.2367x; 2.0889x over previous
import functools

import jax
import jax.numpy as jnp
from jax import lax
from jax.experimental import pallas as pl
from jax.experimental.pallas import tpu as pltpu
from jax.experimental.pallas import tpu_sc as plsc

N = 10000
E = 320000
D = 128
H = 128
C = 40
HB = H // 2

NP = 10240
NT = 32
CH = 64
CHUNKS = 162
E2P = NT * CHUNKS * CH
CHUNKS2 = E2P // (16 * CH)
GB = 2
STRIPE = NP // 16
NHALF = NP // 2
HSTRIPE = NHALF // 16

_mesh = plsc.VectorSubcoreMesh(core_axis_name="c", subcore_axis_name="s",
                               num_cores=2, num_subcores=16)


def _k0_body(x_ref, w_ref, b_ref, h_ref, hn_ref):
    h = jnp.maximum(jnp.dot(x_ref[...], w_ref[...],
                            preferred_element_type=jnp.float32) + b_ref[...], 0.0)
    h_ref[...] = h
    n2 = jnp.sum(h * h, axis=1, keepdims=True)
    hn_ref[...] = h / jnp.maximum(jnp.sqrt(n2), 1e-12)


def _dense_in(x, W1, b1):
    return pl.pallas_call(
        _k0_body,
        out_shape=[jax.ShapeDtypeStruct((NP, H), jnp.float32),
                   jax.ShapeDtypeStruct((NP, H), jnp.float32)],
    )(x, W1, b1.reshape(1, H))


def _comb_body(h_ref, hn_ref):
    h = h_ref[...]
    n2 = jnp.sum(h * h, axis=1, keepdims=True)
    hn_ref[...] = h / jnp.maximum(jnp.sqrt(n2), 1e-12)


def _combine(h):
    return pl.pallas_call(
        _comb_body,
        out_shape=jax.ShapeDtypeStruct((NP, H), jnp.float32),
    )(h)


def _kf_body(h_ref, w_ref, b_ref, o_ref):
    logits = jnp.dot(h_ref[...], w_ref[...],
                     preferred_element_type=jnp.float32) + b_ref[...]
    m = jnp.max(logits, axis=1, keepdims=True)
    z = logits - m
    lse = jnp.log(jnp.sum(jnp.exp(z), axis=1, keepdims=True))
    o_ref[...] = z - lse


def _dense_out(h, W4, b4):
    return pl.pallas_call(
        _kf_body,
        out_shape=jax.ShapeDtypeStruct((NP, C), jnp.float32),
    )(h, W4, b4.reshape(1, C))


def _dsum_body(d_ref, o_ref):
    o_ref[...] = d_ref[0:1, :] + d_ref[1:2, :]


def _den_combine(dens):
    return pl.pallas_call(
        _dsum_body,
        out_shape=jax.ShapeDtypeStruct((1, NP), jnp.float32),
    )(dens)


_P1_KW = dict(
    out_type=[jax.ShapeDtypeStruct((NT, CHUNKS, CH), jnp.float32),
              jax.ShapeDtypeStruct((2, NP), jnp.float32)],
    mesh=_mesh,
    compiler_params=pltpu.CompilerParams(needs_layout_passes=False),
    scratch_types=[
        pltpu.VMEM((CHUNKS, CH), jnp.int32),
        pltpu.VMEM((CHUNKS, CH), jnp.int32),
        pltpu.VMEM((CHUNKS, CH), jnp.float32),
        pltpu.VMEM((CH, H), jnp.float32),
        pltpu.VMEM((CH, H), jnp.float32),
        pltpu.VMEM((16,), jnp.float32),
        pltpu.VMEM((STRIPE,), jnp.float32),
        pltpu.VMEM_SHARED((NP,), jnp.float32),
        pltpu.SemaphoreType.DMA,
        pltpu.SemaphoreType.DMA,
    ],
)


def _sc_pass1_body(hn_hbm, srci_hbm, dsti_hbm, beta_hbm, ex_hbm, den_hbm,
              srci_v, dsti_v, ex_v, bufS, bufT, beta_v, stage_v, den_s,
              semA, semB):
    cid = lax.axis_index("c")
    sid = lax.axis_index("s")
    wid = sid * 2 + cid

    pltpu.sync_copy(srci_hbm.at[wid], srci_v)
    pltpu.sync_copy(dsti_hbm.at[wid], dsti_v)
    pltpu.sync_copy(beta_hbm, beta_v)

    zero16 = jnp.zeros((16,), jnp.float32)
    for i in range(STRIPE // 16):
        stage_v[pl.ds(i * 16, 16)] = zero16
    pltpu.sync_copy(stage_v, den_s.at[pl.ds(sid * STRIPE, STRIPE)])
    plsc.subcore_barrier()

    beta = beta_v[...]
    shift = jnp.abs(beta)
    lane = lax.iota(jnp.int32, 16)

    def chunk(j, carry):
        ga = pltpu.async_copy(hn_hbm.at[srci_v.at[j]], bufS, semA)
        gb = pltpu.async_copy(hn_hbm.at[dsti_v.at[j]], bufT, semB)
        ga.wait()
        gb.wait()
        for g in range(CH // 16):
            rowv = lane + g * 16
            acc = zero16
            for d in range(H):
                cv = jnp.full((16,), d, jnp.int32)
                sv = plsc.load_gather(bufS, [rowv, cv])
                tv = plsc.load_gather(bufT, [rowv, cv])
                acc = acc + sv * tv
            ex_v[j, pl.ds(g * 16, 16)] = jnp.exp(acc * beta - shift)
        pltpu.sync_copy(ex_v.at[j], den_s.at[dsti_v.at[j]], add=True)
        return carry

    lax.fori_loop(0, CHUNKS, chunk, 0)
    plsc.subcore_barrier()

    pltpu.sync_copy(ex_v, ex_hbm.at[wid])
    pltpu.sync_copy(den_s.at[pl.ds(sid * STRIPE, STRIPE)], stage_v)
    pltpu.sync_copy(stage_v, den_hbm.at[cid, pl.ds(sid * STRIPE, STRIPE)])


_P2_KW = dict(
    out_type=jax.ShapeDtypeStruct((NP, H), jnp.float32),
    mesh=_mesh,
    compiler_params=pltpu.CompilerParams(needs_layout_passes=False),
    scratch_types=[
        pltpu.VMEM((GB, CH), jnp.int32),
        pltpu.VMEM((GB, CH), jnp.int32),
        pltpu.VMEM((GB, CH), jnp.int32),
        pltpu.VMEM((GB, CH), jnp.float32),
        pltpu.VMEM((NP,), jnp.float32),
        pltpu.VMEM((CH, H), jnp.float32),
        pltpu.VMEM((CH, H), jnp.float32),
        pltpu.VMEM_SHARED((NHALF + CH, H), jnp.float32),
        pltpu.SemaphoreType.DMA,
    ],
)


def _sc_pass2_body(h_hbm, srci_hbm, dsti_hbm, dsta_hbm, ex_hbm, den_hbm,
                   outp_hbm, srci_v, dsti_v, dsta_v, ex_v, den_v, buf, msg,
                   out_s, semA):
    cid = lax.axis_index("c")
    sid = lax.axis_index("s")
    cbase = cid * NHALF

    pltpu.sync_copy(den_hbm, den_v)

    zero16 = jnp.zeros((16,), jnp.float32)

    def zrow(i, carry):
        for d in range(H // 16):
            msg[i, pl.ds(d * 16, 16)] = zero16
        return carry
    lax.fori_loop(0, CH, zrow, 0)
    for t in range(HSTRIPE // CH):
        pltpu.sync_copy(msg, out_s.at[pl.ds(sid * HSTRIPE + t * CH, CH)])
    plsc.subcore_barrier()

    def group(gi, carry):
        pltpu.sync_copy(srci_hbm.at[sid, pl.ds(gi * GB, GB)], srci_v)
        pltpu.sync_copy(dsti_hbm.at[sid, pl.ds(gi * GB, GB)], dsti_v)
        pltpu.sync_copy(dsta_hbm.at[cid, sid, pl.ds(gi * GB, GB)], dsta_v)
        pltpu.sync_copy(ex_hbm.at[sid, pl.ds(gi * GB, GB)], ex_v)
        for j in range(GB):
            pltpu.async_copy(h_hbm.at[srci_v.at[j]], buf, semA).wait()
            for g in range(CH // 16):
                dst16 = dsti_v[j, pl.ds(g * 16, 16)]
                ex16 = ex_v[j, pl.ds(g * 16, 16)]
                den16 = plsc.load_gather(den_v, [dst16])
                coef16 = ex16 / (den16 + 1e-16)
                for e in range(16):
                    r = g * 16 + e
                    cs = jnp.full((16,), coef16[e])
                    for d in range(H // 16):
                        msg[r, pl.ds(d * 16, 16)] = (
                            buf[r, pl.ds(d * 16, 16)] * cs)
            pltpu.sync_copy(msg, out_s.at[dsta_v.at[j]], add=True)
        return carry

    lax.fori_loop(0, CHUNKS2 // GB, group, 0)
    plsc.subcore_barrier()

    for t in range(HSTRIPE // CH):
        off = sid * HSTRIPE + t * CH
        pltpu.sync_copy(out_s.at[pl.ds(off, CH)], msg)
        pltpu.sync_copy(msg, outp_hbm.at[pl.ds(cbase + off, CH)])


_sc_pass1 = pl.kernel(_sc_pass1_body, **_P1_KW)
_sc_pass2 = pl.kernel(_sc_pass2_body, **_P2_KW)


def kernel(x, edge_index, W1, b1, beta2, beta3, beta5, beta6, W4, b4):
    sl = jnp.arange(N, dtype=jnp.int32)
    npad = E2P - E - N
    src = jnp.concatenate([edge_index[0], sl, jnp.zeros((npad,), jnp.int32)])
    dst = jnp.concatenate([edge_index[1], sl, jnp.full((npad,), N, jnp.int32)])
    srci1 = src.reshape(NT, CHUNKS, CH)
    dsti1 = dst.reshape(NT, CHUNKS, CH)
    srci2 = src.reshape(16, CHUNKS2, CH)
    dsti2 = dst.reshape(16, CHUNKS2, CH)
    dsta2 = jnp.stack([
        jnp.where((dst >= c * NHALF) & (dst < (c + 1) * NHALF),
                  dst - c * NHALF, NHALF)
        for c in (0, 1)]).reshape(2, 16, CHUNKS2, CH)
    xp = jnp.pad(x, ((0, NP - N), (0, 0)))

    h, hn = _dense_in(xp, W1, b1)
    for beta in (beta2, beta3, beta5, beta6):
        beta16 = jnp.broadcast_to(beta, (16,)).astype(jnp.float32)
        ex, dens = _sc_pass1(hn, srci1, dsti1, beta16)
        den = _den_combine(dens).reshape(NP)
        h = _sc_pass2(h, srci2, dsti2, dsta2,
                      ex.reshape(16, CHUNKS2, CH), den)
        if beta is not beta6:
            hn = _combine(h)
    out = _dense_out(h, W4, b4)
    return out[:N]


# speedup vs baseline: 3.3292x; 1.0286x over previous
import functools

import jax
import jax.numpy as jnp
from jax import lax
from jax.experimental import pallas as pl
from jax.experimental.pallas import tpu as pltpu
from jax.experimental.pallas import tpu_sc as plsc

N = 10000
E = 320000
D = 128
H = 128
C = 40
HB = H // 2

NP = 10240
NT = 32
CH = 64
CHUNKS = 162
E2P = NT * CHUNKS * CH
CHUNKS2 = E2P // (16 * CH)
GB = 4
STRIPE = NP // 16
NHALF = NP // 2
HSTRIPE = NHALF // 16

_mesh = plsc.VectorSubcoreMesh(core_axis_name="c", subcore_axis_name="s",
                               num_cores=2, num_subcores=16)


def _k0_body(x_ref, w_ref, b_ref, h_ref, hn_ref):
    h = jnp.maximum(jnp.dot(x_ref[...], w_ref[...],
                            preferred_element_type=jnp.float32) + b_ref[...], 0.0)
    h_ref[...] = h
    n2 = jnp.sum(h * h, axis=1, keepdims=True)
    hn_ref[...] = h / jnp.maximum(jnp.sqrt(n2), 1e-12)


def _dense_in(x, W1, b1):
    return pl.pallas_call(
        _k0_body,
        out_shape=[jax.ShapeDtypeStruct((NP, H), jnp.float32),
                   jax.ShapeDtypeStruct((NP, H), jnp.float32)],
    )(x, W1, b1.reshape(1, H))


def _comb_body(h_ref, hn_ref):
    h = h_ref[...]
    n2 = jnp.sum(h * h, axis=1, keepdims=True)
    hn_ref[...] = h / jnp.maximum(jnp.sqrt(n2), 1e-12)


def _combine(h):
    return pl.pallas_call(
        _comb_body,
        out_shape=jax.ShapeDtypeStruct((NP, H), jnp.float32),
    )(h)


def _kf_body(h_ref, w_ref, b_ref, o_ref):
    logits = jnp.dot(h_ref[...], w_ref[...],
                     preferred_element_type=jnp.float32) + b_ref[...]
    m = jnp.max(logits, axis=1, keepdims=True)
    z = logits - m
    lse = jnp.log(jnp.sum(jnp.exp(z), axis=1, keepdims=True))
    o_ref[...] = z - lse


def _dense_out(h, W4, b4):
    return pl.pallas_call(
        _kf_body,
        out_shape=jax.ShapeDtypeStruct((NP, C), jnp.float32),
    )(h, W4, b4.reshape(1, C))


def _dsum_body(d_ref, o_ref):
    o_ref[...] = d_ref[0:1, :] + d_ref[1:2, :]


def _den_combine(dens):
    return pl.pallas_call(
        _dsum_body,
        out_shape=jax.ShapeDtypeStruct((1, NP), jnp.float32),
    )(dens)


_P1_KW = dict(
    out_type=[jax.ShapeDtypeStruct((NT, CHUNKS, CH), jnp.float32),
              jax.ShapeDtypeStruct((2, NP), jnp.float32)],
    mesh=_mesh,
    compiler_params=pltpu.CompilerParams(needs_layout_passes=False),
    scratch_types=[
        pltpu.VMEM((CHUNKS, CH), jnp.int32),
        pltpu.VMEM((CHUNKS, CH), jnp.int32),
        pltpu.VMEM((CHUNKS, CH), jnp.float32),
        pltpu.VMEM((CH, H), jnp.float32),
        pltpu.VMEM((CH, H), jnp.float32),
        pltpu.VMEM((CH, H), jnp.float32),
        pltpu.VMEM((CH, H), jnp.float32),
        pltpu.VMEM((16,), jnp.float32),
        pltpu.VMEM((STRIPE,), jnp.float32),
        pltpu.VMEM_SHARED((NP,), jnp.float32),
        pltpu.SemaphoreType.DMA,
        pltpu.SemaphoreType.DMA,
        pltpu.SemaphoreType.DMA,
        pltpu.SemaphoreType.DMA,
        pltpu.SemaphoreType.DMA,
    ],
)


def _sc_pass1_body(hn_hbm, srci_hbm, dsti_hbm, beta_hbm, ex_hbm, den_hbm,
                   srci_v, dsti_v, ex_v, bufS0, bufT0, bufS1, bufT1, beta_v,
                   stage_v, den_s, semA0, semB0, semA1, semB1, semS):
    cid = lax.axis_index("c")
    sid = lax.axis_index("s")
    wid = sid * 2 + cid

    pltpu.sync_copy(srci_hbm.at[wid], srci_v)
    pltpu.sync_copy(dsti_hbm.at[wid], dsti_v)
    pltpu.sync_copy(beta_hbm, beta_v)

    zero16 = jnp.zeros((16,), jnp.float32)
    for i in range(STRIPE // 16):
        stage_v[pl.ds(i * 16, 16)] = zero16
    pltpu.sync_copy(stage_v, den_s.at[pl.ds(sid * STRIPE, STRIPE)])
    plsc.subcore_barrier()

    beta = beta_v[...]
    shift = jnp.abs(beta)
    lane = lax.iota(jnp.int32, 16)

    def compute(j, bufS, bufT):
        for g in range(CH // 16):
            rowv = lane + g * 16
            a0 = zero16
            a1 = zero16
            a2 = zero16
            a3 = zero16
            for d in range(0, H, 4):
                a0 = a0 + (plsc.load_gather(bufS, [rowv, jnp.full((16,), d, jnp.int32)])
                           * plsc.load_gather(bufT, [rowv, jnp.full((16,), d, jnp.int32)]))
                a1 = a1 + (plsc.load_gather(bufS, [rowv, jnp.full((16,), d + 1, jnp.int32)])
                           * plsc.load_gather(bufT, [rowv, jnp.full((16,), d + 1, jnp.int32)]))
                a2 = a2 + (plsc.load_gather(bufS, [rowv, jnp.full((16,), d + 2, jnp.int32)])
                           * plsc.load_gather(bufT, [rowv, jnp.full((16,), d + 2, jnp.int32)]))
                a3 = a3 + (plsc.load_gather(bufS, [rowv, jnp.full((16,), d + 3, jnp.int32)])
                           * plsc.load_gather(bufT, [rowv, jnp.full((16,), d + 3, jnp.int32)]))
            acc = (a0 + a1) + (a2 + a3)
            ex_v[j, pl.ds(g * 16, 16)] = jnp.exp(acc * beta - shift)
        pltpu.sync_copy(ex_v.at[j], den_s.at[dsti_v.at[j]], add=True)

    d0a = pltpu.async_copy(hn_hbm.at[srci_v.at[0]], bufS0, semA0)
    d0b = pltpu.async_copy(hn_hbm.at[dsti_v.at[0]], bufT0, semB0)

    def pair(k, carry):
        a = 2 * k
        b = 2 * k + 1
        nxt = (2 * k + 2) % CHUNKS
        pltpu.async_copy(hn_hbm.at[srci_v.at[b]], bufS1, semA1)
        pltpu.async_copy(hn_hbm.at[dsti_v.at[b]], bufT1, semB1)
        pltpu.make_async_copy(hn_hbm.at[srci_v.at[a]], bufS0, semA0).wait()
        pltpu.make_async_copy(hn_hbm.at[dsti_v.at[a]], bufT0, semB0).wait()
        compute(a, bufS0, bufT0)
        pltpu.async_copy(hn_hbm.at[srci_v.at[nxt]], bufS0, semA0)
        pltpu.async_copy(hn_hbm.at[dsti_v.at[nxt]], bufT0, semB0)
        pltpu.make_async_copy(hn_hbm.at[srci_v.at[b]], bufS1, semA1).wait()
        pltpu.make_async_copy(hn_hbm.at[dsti_v.at[b]], bufT1, semB1).wait()
        compute(b, bufS1, bufT1)
        return carry

    lax.fori_loop(0, CHUNKS // 2, pair, 0)

    pltpu.make_async_copy(hn_hbm.at[srci_v.at[0]], bufS0, semA0).wait()
    pltpu.make_async_copy(hn_hbm.at[dsti_v.at[0]], bufT0, semB0).wait()
    plsc.subcore_barrier()

    pltpu.sync_copy(ex_v, ex_hbm.at[wid])
    pltpu.sync_copy(den_s.at[pl.ds(sid * STRIPE, STRIPE)], stage_v)
    pltpu.sync_copy(stage_v, den_hbm.at[cid, pl.ds(sid * STRIPE, STRIPE)])


_P2_KW = dict(
    out_type=jax.ShapeDtypeStruct((NP, H), jnp.float32),
    mesh=_mesh,
    compiler_params=pltpu.CompilerParams(needs_layout_passes=False),
    scratch_types=[
        pltpu.VMEM((GB, CH), jnp.int32),
        pltpu.VMEM((GB, CH), jnp.int32),
        pltpu.VMEM((GB, CH), jnp.int32),
        pltpu.VMEM((GB, CH), jnp.float32),
        pltpu.VMEM((NP,), jnp.float32),
        pltpu.VMEM((CH, H), jnp.float32),
        pltpu.VMEM((CH, H), jnp.float32),
        pltpu.VMEM((CH, H), jnp.float32),
        pltpu.VMEM((CH, H), jnp.float32),
        pltpu.VMEM_SHARED((NHALF + CH, H), jnp.float32),
        pltpu.SemaphoreType.DMA,
        pltpu.SemaphoreType.DMA,
        pltpu.SemaphoreType.DMA,
        pltpu.SemaphoreType.DMA,
    ],
)


def _sc_pass2_body(h_hbm, srci_hbm, dsti_hbm, dsta_hbm, ex_hbm, den_hbm,
                   outp_hbm, srci_v, dsti_v, dsta_v, ex_v, den_v, bufA, bufB,
                   msgA, msgB, out_s, semGA, semGB, semSA, semSB):
    cid = lax.axis_index("c")
    sid = lax.axis_index("s")
    cbase = cid * NHALF

    pltpu.sync_copy(den_hbm, den_v)

    zero16 = jnp.zeros((16,), jnp.float32)

    def zrow(i, carry):
        for d in range(H // 16):
            msgA[i, pl.ds(d * 16, 16)] = zero16
            msgB[i, pl.ds(d * 16, 16)] = zero16
        return carry
    lax.fori_loop(0, CH, zrow, 0)
    for t in range(HSTRIPE // CH):
        pltpu.sync_copy(msgA, out_s.at[pl.ds(sid * HSTRIPE + t * CH, CH)])
    plsc.subcore_barrier()

    def compute(j, buf, msg):
        for g in range(CH // 16):
            dst16 = dsti_v[j, pl.ds(g * 16, 16)]
            ex16 = ex_v[j, pl.ds(g * 16, 16)]
            den16 = plsc.load_gather(den_v, [dst16])
            coef16 = ex16 / (den16 + 1e-16)
            for e in range(16):
                r = g * 16 + e
                cs = jnp.full((16,), coef16[e])
                for d in range(H // 16):
                    msg[r, pl.ds(d * 16, 16)] = (
                        buf[r, pl.ds(d * 16, 16)] * cs)

    def group(gi, carry):
        base = gi * GB
        pltpu.sync_copy(srci_hbm.at[sid, pl.ds(base, GB)], srci_v)
        pltpu.sync_copy(dsti_hbm.at[sid, pl.ds(base, GB)], dsti_v)
        pltpu.sync_copy(dsta_hbm.at[cid, sid, pl.ds(base, GB)], dsta_v)
        pltpu.sync_copy(ex_hbm.at[sid, pl.ds(base, GB)], ex_v)
        pltpu.async_copy(h_hbm.at[srci_v.at[0]], bufA, semGA)
        for jj in range(GB):
            buf, msg, semG, semS = ((bufA, msgA, semGA, semSA) if jj % 2 == 0
                                    else (bufB, msgB, semGB, semSB))
            if jj + 1 < GB:
                nbuf, nsem = (bufB, semGB) if jj % 2 == 0 else (bufA, semGA)
                pltpu.async_copy(h_hbm.at[srci_v.at[jj + 1]], nbuf, nsem)
            pltpu.make_async_copy(h_hbm.at[srci_v.at[jj]], buf, semG).wait()
            compute(jj, buf, msg)
            pltpu.sync_copy(msg, out_s.at[dsta_v.at[jj]], add=True)
        return carry

    lax.fori_loop(0, CHUNKS2 // GB, group, 0)
    plsc.subcore_barrier()

    for t in range(HSTRIPE // CH):
        off = sid * HSTRIPE + t * CH
        pltpu.sync_copy(out_s.at[pl.ds(off, CH)], msgA)
        pltpu.sync_copy(msgA, outp_hbm.at[pl.ds(cbase + off, CH)])


_sc_pass1 = pl.kernel(_sc_pass1_body, **_P1_KW)
_sc_pass2 = pl.kernel(_sc_pass2_body, **_P2_KW)


def kernel(x, edge_index, W1, b1, beta2, beta3, beta5, beta6, W4, b4):
    sl = jnp.arange(N, dtype=jnp.int32)
    npad = E2P - E - N
    src = jnp.concatenate([edge_index[0], sl, jnp.zeros((npad,), jnp.int32)])
    dst = jnp.concatenate([edge_index[1], sl, jnp.full((npad,), N, jnp.int32)])
    srci1 = src.reshape(NT, CHUNKS, CH)
    dsti1 = dst.reshape(NT, CHUNKS, CH)
    srci2 = src.reshape(16, CHUNKS2, CH)
    dsti2 = dst.reshape(16, CHUNKS2, CH)
    dsta2 = jnp.stack([
        jnp.where((dst >= c * NHALF) & (dst < (c + 1) * NHALF),
                  dst - c * NHALF, NHALF)
        for c in (0, 1)]).reshape(2, 16, CHUNKS2, CH)
    xp = jnp.pad(x, ((0, NP - N), (0, 0)))

    h, hn = _dense_in(xp, W1, b1)
    for beta in (beta2, beta3, beta5, beta6):
        beta16 = jnp.broadcast_to(beta, (16,)).astype(jnp.float32)
        ex, dens = _sc_pass1(hn, srci1, dsti1, beta16)
        den = _den_combine(dens).reshape(NP)
        h = _sc_pass2(h, srci2, dsti2, dsta2,
                      ex.reshape(16, CHUNKS2, CH), den)
        if beta is not beta6:
            hn = _combine(h)
    out = _dense_out(h, W4, b4)
    return out[:N]


# speedup vs baseline: 3.4058x; 1.0230x over previous
import functools

import jax
import jax.numpy as jnp
from jax import lax
from jax.experimental import pallas as pl
from jax.experimental.pallas import tpu as pltpu
from jax.experimental.pallas import tpu_sc as plsc

N = 10000
E = 320000
D = 128
H = 128
C = 40
HB = H // 2

NP = 10240
NT = 32
CH = 64
CHUNKS = 162
E2P = NT * CHUNKS * CH
CHUNKS2 = E2P // (16 * CH)
GB = 4
STRIPE = NP // 16
NHALF = NP // 2
HSTRIPE = NHALF // 16

_mesh = plsc.VectorSubcoreMesh(core_axis_name="c", subcore_axis_name="s",
                               num_cores=2, num_subcores=16)


def _k0_body(x_ref, w_ref, b_ref, h_ref, hn_ref):
    h = jnp.maximum(jnp.dot(x_ref[...], w_ref[...],
                            preferred_element_type=jnp.float32) + b_ref[...], 0.0)
    h_ref[...] = h
    n2 = jnp.sum(h * h, axis=1, keepdims=True)
    hn_ref[...] = h / jnp.maximum(jnp.sqrt(n2), 1e-12)


def _dense_in(x, W1, b1):
    return pl.pallas_call(
        _k0_body,
        out_shape=[jax.ShapeDtypeStruct((NP, H), jnp.float32),
                   jax.ShapeDtypeStruct((NP, H), jnp.float32)],
    )(x, W1, b1.reshape(1, H))


def _comb_body(h_ref, hn_ref):
    h = h_ref[...]
    n2 = jnp.sum(h * h, axis=1, keepdims=True)
    hn_ref[...] = h / jnp.maximum(jnp.sqrt(n2), 1e-12)


def _combine(h):
    return pl.pallas_call(
        _comb_body,
        out_shape=jax.ShapeDtypeStruct((NP, H), jnp.float32),
    )(h)


def _kf_body(h_ref, w_ref, b_ref, o_ref):
    logits = jnp.dot(h_ref[...], w_ref[...],
                     preferred_element_type=jnp.float32) + b_ref[...]
    m = jnp.max(logits, axis=1, keepdims=True)
    z = logits - m
    lse = jnp.log(jnp.sum(jnp.exp(z), axis=1, keepdims=True))
    o_ref[...] = z - lse


def _dense_out(h, W4, b4):
    return pl.pallas_call(
        _kf_body,
        out_shape=jax.ShapeDtypeStruct((NP, C), jnp.float32),
    )(h, W4, b4.reshape(1, C))


def _dsum_body(d_ref, o_ref):
    o_ref[...] = d_ref[0:1, :] + d_ref[1:2, :]


def _den_combine(dens):
    return pl.pallas_call(
        _dsum_body,
        out_shape=jax.ShapeDtypeStruct((1, NP), jnp.float32),
    )(dens)


_P1_KW = dict(
    out_type=[jax.ShapeDtypeStruct((NT, CHUNKS, CH), jnp.float32),
              jax.ShapeDtypeStruct((2, NP), jnp.float32)],
    mesh=_mesh,
    compiler_params=pltpu.CompilerParams(needs_layout_passes=False),
    scratch_types=[
        pltpu.VMEM((CHUNKS, CH), jnp.int32),
        pltpu.VMEM((CHUNKS, CH), jnp.int32),
        pltpu.VMEM((CHUNKS, CH), jnp.float32),
        pltpu.VMEM((CH, H), jnp.float32),
        pltpu.VMEM((CH, H), jnp.float32),
        pltpu.VMEM((CH, H), jnp.float32),
        pltpu.VMEM((CH, H), jnp.float32),
        pltpu.VMEM((16,), jnp.float32),
        pltpu.VMEM((STRIPE,), jnp.float32),
        pltpu.VMEM_SHARED((NP,), jnp.float32),
        pltpu.SemaphoreType.DMA,
        pltpu.SemaphoreType.DMA,
        pltpu.SemaphoreType.DMA,
        pltpu.SemaphoreType.DMA,
        pltpu.SemaphoreType.DMA,
    ],
)


def _sc_pass1_body(hn_hbm, srci_hbm, dsti_hbm, beta_hbm, ex_hbm, den_hbm,
                   srci_v, dsti_v, ex_v, bufS0, bufT0, bufS1, bufT1, beta_v,
                   stage_v, den_s, semA0, semB0, semA1, semB1, semS):
    cid = lax.axis_index("c")
    sid = lax.axis_index("s")
    wid = sid * 2 + cid

    pltpu.sync_copy(srci_hbm.at[wid], srci_v)
    pltpu.sync_copy(dsti_hbm.at[wid], dsti_v)
    pltpu.sync_copy(beta_hbm, beta_v)

    zero16 = jnp.zeros((16,), jnp.float32)
    for i in range(STRIPE // 16):
        stage_v[pl.ds(i * 16, 16)] = zero16
    pltpu.sync_copy(stage_v, den_s.at[pl.ds(sid * STRIPE, STRIPE)])
    plsc.subcore_barrier()

    beta = beta_v[...]
    shift = jnp.abs(beta)
    lane = lax.iota(jnp.int32, 16)

    def compute(j, bufS, bufT):
        for g in range(CH // 16):
            rowv = lane + g * 16
            a0 = zero16
            a1 = zero16
            a2 = zero16
            a3 = zero16
            for d in range(0, H, 4):
                a0 = a0 + (plsc.load_gather(bufS, [rowv, jnp.full((16,), d, jnp.int32)])
                           * plsc.load_gather(bufT, [rowv, jnp.full((16,), d, jnp.int32)]))
                a1 = a1 + (plsc.load_gather(bufS, [rowv, jnp.full((16,), d + 1, jnp.int32)])
                           * plsc.load_gather(bufT, [rowv, jnp.full((16,), d + 1, jnp.int32)]))
                a2 = a2 + (plsc.load_gather(bufS, [rowv, jnp.full((16,), d + 2, jnp.int32)])
                           * plsc.load_gather(bufT, [rowv, jnp.full((16,), d + 2, jnp.int32)]))
                a3 = a3 + (plsc.load_gather(bufS, [rowv, jnp.full((16,), d + 3, jnp.int32)])
                           * plsc.load_gather(bufT, [rowv, jnp.full((16,), d + 3, jnp.int32)]))
            acc = (a0 + a1) + (a2 + a3)
            ex_v[j, pl.ds(g * 16, 16)] = jnp.exp(acc * beta - shift)
        pltpu.sync_copy(ex_v.at[j], den_s.at[dsti_v.at[j]], add=True)

    d0a = pltpu.async_copy(hn_hbm.at[srci_v.at[0]], bufS0, semA0)
    d0b = pltpu.async_copy(hn_hbm.at[dsti_v.at[0]], bufT0, semB0)

    def pair(k, carry):
        a = 2 * k
        b = 2 * k + 1
        nxt = (2 * k + 2) % CHUNKS
        pltpu.async_copy(hn_hbm.at[srci_v.at[b]], bufS1, semA1)
        pltpu.async_copy(hn_hbm.at[dsti_v.at[b]], bufT1, semB1)
        pltpu.make_async_copy(hn_hbm.at[srci_v.at[a]], bufS0, semA0).wait()
        pltpu.make_async_copy(hn_hbm.at[dsti_v.at[a]], bufT0, semB0).wait()
        compute(a, bufS0, bufT0)
        pltpu.async_copy(hn_hbm.at[srci_v.at[nxt]], bufS0, semA0)
        pltpu.async_copy(hn_hbm.at[dsti_v.at[nxt]], bufT0, semB0)
        pltpu.make_async_copy(hn_hbm.at[srci_v.at[b]], bufS1, semA1).wait()
        pltpu.make_async_copy(hn_hbm.at[dsti_v.at[b]], bufT1, semB1).wait()
        compute(b, bufS1, bufT1)
        return carry

    lax.fori_loop(0, CHUNKS // 2, pair, 0)

    pltpu.make_async_copy(hn_hbm.at[srci_v.at[0]], bufS0, semA0).wait()
    pltpu.make_async_copy(hn_hbm.at[dsti_v.at[0]], bufT0, semB0).wait()
    plsc.subcore_barrier()

    pltpu.sync_copy(ex_v, ex_hbm.at[wid])
    pltpu.sync_copy(den_s.at[pl.ds(sid * STRIPE, STRIPE)], stage_v)
    pltpu.sync_copy(stage_v, den_hbm.at[cid, pl.ds(sid * STRIPE, STRIPE)])


_P2_KW = dict(
    out_type=jax.ShapeDtypeStruct((NP, H), jnp.float32),
    mesh=_mesh,
    compiler_params=pltpu.CompilerParams(needs_layout_passes=False),
    scratch_types=[
        pltpu.VMEM((GB, CH), jnp.int32),
        pltpu.VMEM((GB, CH), jnp.int32),
        pltpu.VMEM((GB, CH), jnp.int32),
        pltpu.VMEM((GB, CH), jnp.float32),
        pltpu.VMEM((NP,), jnp.float32),
        pltpu.VMEM((CH, H), jnp.float32),
        pltpu.VMEM((CH, H), jnp.float32),
        pltpu.VMEM((CH, H), jnp.float32),
        pltpu.VMEM((CH, H), jnp.float32),
        pltpu.VMEM_SHARED((NHALF + CH, H), jnp.float32),
        pltpu.SemaphoreType.DMA,
        pltpu.SemaphoreType.DMA,
        pltpu.SemaphoreType.DMA,
        pltpu.SemaphoreType.DMA,
    ],
)


def _sc_pass2_body(h_hbm, srci_hbm, dsti_hbm, dsta_hbm, ex_hbm, den_hbm,
                   outp_hbm, srci_v, dsti_v, dsta_v, ex_v, den_v, bufA, bufB,
                   msgA, msgB, out_s, semGA, semGB, semSA, semSB):
    cid = lax.axis_index("c")
    sid = lax.axis_index("s")
    cbase = cid * NHALF

    pltpu.sync_copy(den_hbm, den_v)

    zero16 = jnp.zeros((16,), jnp.float32)

    def zrow(i, carry):
        for d in range(H // 16):
            msgA[i, pl.ds(d * 16, 16)] = zero16
            msgB[i, pl.ds(d * 16, 16)] = zero16
        return carry
    lax.fori_loop(0, CH, zrow, 0)
    for t in range(HSTRIPE // CH):
        pltpu.sync_copy(msgA, out_s.at[pl.ds(sid * HSTRIPE + t * CH, CH)])
    plsc.subcore_barrier()

    pltpu.async_copy(msgA, out_s.at[pl.ds(NHALF, CH)], semSA)
    pltpu.async_copy(msgB, out_s.at[pl.ds(NHALF, CH)], semSB)

    def compute(j, buf, msg):
        for g in range(CH // 16):
            dst16 = dsti_v[j, pl.ds(g * 16, 16)]
            ex16 = ex_v[j, pl.ds(g * 16, 16)]
            den16 = plsc.load_gather(den_v, [dst16])
            coef16 = ex16 / (den16 + 1e-16)
            for e in range(16):
                r = g * 16 + e
                cs = jnp.full((16,), coef16[e])
                for d in range(H // 16):
                    msg[r, pl.ds(d * 16, 16)] = (
                        buf[r, pl.ds(d * 16, 16)] * cs)

    def group(gi, carry):
        base = gi * GB
        pltpu.sync_copy(srci_hbm.at[sid, pl.ds(base, GB)], srci_v)
        pltpu.sync_copy(dsti_hbm.at[sid, pl.ds(base, GB)], dsti_v)
        pltpu.sync_copy(dsta_hbm.at[cid, sid, pl.ds(base, GB)], dsta_v)
        pltpu.sync_copy(ex_hbm.at[sid, pl.ds(base, GB)], ex_v)
        pltpu.async_copy(h_hbm.at[srci_v.at[0]], bufA, semGA)
        for jj in range(GB):
            buf, msg, semG, semS = ((bufA, msgA, semGA, semSA) if jj % 2 == 0
                                    else (bufB, msgB, semGB, semSB))
            if jj + 1 < GB:
                nbuf, nsem = (bufB, semGB) if jj % 2 == 0 else (bufA, semGA)
                pltpu.async_copy(h_hbm.at[srci_v.at[jj + 1]], nbuf, nsem)
            pltpu.make_async_copy(h_hbm.at[srci_v.at[jj]], buf, semG).wait()
            pltpu.make_async_copy(msg, out_s.at[pl.ds(NHALF, CH)], semS).wait()
            compute(jj, buf, msg)
            pltpu.async_copy(msg, out_s.at[dsta_v.at[jj]], semS, add=True)
        return carry

    lax.fori_loop(0, CHUNKS2 // GB, group, 0)

    pltpu.make_async_copy(msgA, out_s.at[pl.ds(NHALF, CH)], semSA).wait()
    pltpu.make_async_copy(msgB, out_s.at[pl.ds(NHALF, CH)], semSB).wait()
    plsc.subcore_barrier()

    for t in range(HSTRIPE // CH):
        off = sid * HSTRIPE + t * CH
        pltpu.sync_copy(out_s.at[pl.ds(off, CH)], msgA)
        pltpu.sync_copy(msgA, outp_hbm.at[pl.ds(cbase + off, CH)])


_sc_pass1 = pl.kernel(_sc_pass1_body, **_P1_KW)
_sc_pass2 = pl.kernel(_sc_pass2_body, **_P2_KW)


def kernel(x, edge_index, W1, b1, beta2, beta3, beta5, beta6, W4, b4):
    sl = jnp.arange(N, dtype=jnp.int32)
    npad = E2P - E - N
    src = jnp.concatenate([edge_index[0], sl, jnp.zeros((npad,), jnp.int32)])
    dst = jnp.concatenate([edge_index[1], sl, jnp.full((npad,), N, jnp.int32)])
    srci1 = src.reshape(NT, CHUNKS, CH)
    dsti1 = dst.reshape(NT, CHUNKS, CH)
    srci2 = src.reshape(16, CHUNKS2, CH)
    dsti2 = dst.reshape(16, CHUNKS2, CH)
    dsta2 = jnp.stack([
        jnp.where((dst >= c * NHALF) & (dst < (c + 1) * NHALF),
                  dst - c * NHALF, NHALF)
        for c in (0, 1)]).reshape(2, 16, CHUNKS2, CH)
    xp = jnp.pad(x, ((0, NP - N), (0, 0)))

    h, hn = _dense_in(xp, W1, b1)
    for beta in (beta2, beta3, beta5, beta6):
        beta16 = jnp.broadcast_to(beta, (16,)).astype(jnp.float32)
        ex, dens = _sc_pass1(hn, srci1, dsti1, beta16)
        den = _den_combine(dens).reshape(NP)
        h = _sc_pass2(h, srci2, dsti2, dsta2,
                      ex.reshape(16, CHUNKS2, CH), den)
        if beta is not beta6:
            hn = _combine(h)
    out = _dense_out(h, W4, b4)
    return out[:N]
